# Initial kernel scaffold; baseline (speedup 1.0000x reference)
import jax
import jax.numpy as jnp
from jax.experimental import pallas as pl


def kernel(x, edge_index, W_gc, b_gc, Wsrc0, bsrc0, Wdst0, bdst0, attn0, Wsrc1, bsrc1, Wdst1, bdst1, attn1, Wsrc2, bsrc2, Wdst2, bdst2, attn2, Wc1, bc1, Wc2, bc2, Wc3, bc3):
    raise NotImplementedError("write your pallas kernel here")


# speedup vs baseline: 15.5970x; 15.5970x over previous
import functools

import jax
import jax.numpy as jnp
from jax import lax
from jax.experimental import pallas as pl
from jax.experimental.pallas import tpu as pltpu
from jax.experimental.pallas import tpu_sc as plsc

N = 10000
E = 320000
D = 128
H = 8
DH = 16
OUT = 64

NC = 2
NS = 16
NW = NC * NS
L = 16
EPW = E // NW
NP = 10112
PPT = NP // NS
DA = D + L

CA = 80
NCHA = EPW // CA
CG = 40
CGB = 48
NCHG = EPW // CG

_mesh = plsc.VectorSubcoreMesh(core_axis_name="c", subcore_axis_name="s")
_f32 = jnp.float32
_sc_params = pltpu.CompilerParams(needs_layout_passes=False,
                                  use_tc_tiling_on_sc=False)
_MASK16 = 0xFFFF


def _unpack(pv):
    return jnp.bitwise_and(pv, _MASK16), lax.shift_right_logical(pv, 16)


@functools.partial(
    pl.kernel, mesh=_mesh,
    out_type=(jax.ShapeDtypeStruct((NW * N,), _f32),
              jax.ShapeDtypeStruct((NW * N,), _f32)),
    scratch_types=[pltpu.VMEM((EPW,), jnp.int32),
                   pltpu.VMEM((N,), _f32),
                   pltpu.VMEM((N,), _f32)],
    compiler_params=_sc_params,
)
def _sc_degrees(pk_hbm, outs_hbm, outd_hbm, pk_v, hs_v, hd_v):
    cid = lax.axis_index("c")
    sid = lax.axis_index("s")
    wid = sid * NC + cid
    zero16 = jnp.zeros((L,), _f32)

    def zb(i, _):
        hs_v[pl.ds(i * L, L)] = zero16
        hd_v[pl.ds(i * L, L)] = zero16
        return 0

    lax.fori_loop(0, N // L, zb, 0)
    pltpu.sync_copy(pk_hbm.at[pl.ds(wid * EPW, EPW)], pk_v)
    ones16 = jnp.ones((L,), _f32)

    def body(i, _):
        si, di = _unpack(pk_v[pl.ds(i * L, L)])
        plsc.addupdate_scatter(hs_v, [si], ones16)
        plsc.addupdate_scatter(hd_v, [di], ones16)
        return 0

    lax.fori_loop(0, EPW // L, body, 0)
    pltpu.sync_copy(hs_v, outs_hbm.at[pl.ds(wid * N, N)])
    pltpu.sync_copy(hd_v, outd_hbm.at[pl.ds(wid * N, N)])


@functools.partial(
    pl.kernel, mesh=_mesh,
    out_type=jax.ShapeDtypeStruct((NC, NS, PPT, D), _f32),
    scratch_types=[pltpu.VMEM((2, CA), jnp.int32),
                   pltpu.VMEM((2, CA), jnp.int32),
                   pltpu.VMEM((2, CA), jnp.int32),
                   pltpu.VMEM((2, CA, D), _f32),
                   pltpu.VMEM_SHARED((NP, D), _f32),
                   pltpu.SemaphoreType.DMA((2,)),
                   pltpu.SemaphoreType.DMA((2,)),
                   pltpu.SemaphoreType.DMA((2,))],
    compiler_params=_sc_params,
)
def _sc_gc_agg(xwn_hbm, pk_hbm, out_hbm, pk2, si2, di2, rows2, acc_s,
               s_i, s_g, s_s):
    cid = lax.axis_index("c")
    sid = lax.axis_index("s")
    wid = sid * NC + cid
    ebase = wid * EPW
    zero16 = jnp.zeros((L,), _f32)

    def zb(i, _):
        rows2[0, i // (D // L), pl.ds((i % (D // L)) * L, L)] = zero16
        return 0

    lax.fori_loop(0, CA * (D // L), zb, 0)
    nbase = sid * PPT
    for k in range(PPT // CA):
        pltpu.sync_copy(rows2.at[0], acc_s.at[pl.ds(nbase + k * CA, CA)])
    rem = PPT - (PPT // CA) * CA
    pltpu.sync_copy(rows2.at[0, pl.ds(0, rem)],
                    acc_s.at[pl.ds(nbase + PPT - rem, rem)])
    plsc.subcore_barrier()

    def unpack_chunk(slot):
        for r in range(CA // L):
            si, di = _unpack(pk2[slot, pl.ds(r * L, L)])
            si2[slot, pl.ds(r * L, L)] = si
            di2[slot, pl.ds(r * L, L)] = di

    def idx_start(c, slot):
        pltpu.async_copy(pk_hbm.at[pl.ds(ebase + c * CA, CA)],
                         pk2.at[slot], s_i.at[slot])

    def idx_wait(slot):
        pltpu.make_async_copy(pk_hbm.at[pl.ds(ebase, CA)], pk2.at[slot],
                              s_i.at[slot]).wait()

    def gather_start(slot):
        pltpu.async_copy(xwn_hbm.at[si2.at[slot]], rows2.at[slot],
                         s_g.at[slot])

    def gather_wait(slot):
        pltpu.make_async_copy(xwn_hbm.at[si2.at[slot]], rows2.at[slot],
                              s_g.at[slot]).wait()

    def scat_start(slot):
        pltpu.async_copy(rows2.at[slot], acc_s.at[di2.at[slot]],
                         s_s.at[slot], add=True)

    def scat_wait(slot):
        pltpu.make_async_copy(rows2.at[slot], acc_s.at[di2.at[slot]],
                              s_s.at[slot]).wait()

    idx_start(0, 0)
    idx_wait(0)
    unpack_chunk(0)
    gather_start(0)
    idx_start(1, 1)

    def chunk(c, _):
        b = jnp.bitwise_and(c, 1)
        nb = 1 - b

        @pl.when(c + 1 < NCHA)
        def _():
            idx_wait(nb)

            @pl.when(c >= 1)
            def _():
                scat_wait(nb)

            unpack_chunk(nb)
            gather_start(nb)

            @pl.when(c + 2 < NCHA)
            def _():
                idx_start(c + 2, b)

        gather_wait(b)
        scat_start(b)
        return 0

    lax.fori_loop(0, NCHA, chunk, 0)
    scat_wait((NCHA - 1) & 1)
    scat_wait((NCHA - 2) & 1)
    plsc.subcore_barrier()
    pltpu.sync_copy(acc_s.at[pl.ds(nbase, PPT)], out_hbm.at[cid, sid])


@functools.partial(
    pl.kernel, mesh=_mesh,
    out_type=jax.ShapeDtypeStruct((NC, NS, PPT, DA), _f32),
    scratch_types=[pltpu.VMEM((2, CGB), jnp.int32),
                   pltpu.VMEM((2, CGB), jnp.int32),
                   pltpu.VMEM((2, CGB), jnp.int32),
                   pltpu.VMEM((2, CGB, 2 * D), _f32),
                   pltpu.VMEM((2, CGB, DA), _f32),
                   pltpu.VMEM((H, L), _f32),
                   pltpu.VMEM_SHARED((NP, DA), _f32),
                   pltpu.SemaphoreType.DMA((2,)),
                   pltpu.SemaphoreType.DMA((2,)),
                   pltpu.SemaphoreType.DMA((2,))],
    compiler_params=_sc_params,
)
def _sc_gat(ff_hbm, attn_hbm, pk_hbm, out_hbm, pk2, si2, di2, fb2, msg2,
            attn_v, acc_s, s_i, s_g, s_s):
    cid = lax.axis_index("c")
    sid = lax.axis_index("s")
    wid = sid * NC + cid
    ebase = wid * EPW
    pltpu.sync_copy(attn_hbm, attn_v)
    zero16 = jnp.zeros((L,), _f32)
    lane = lax.iota(jnp.int32, L)
    m8 = lane < 8

    def zb2(i, _):
        s = i // (CGB * (DA // L))
        r = (i // (DA // L)) % CGB
        msg2[s, r, pl.ds((i % (DA // L)) * L, L)] = zero16
        return 0

    lax.fori_loop(0, 2 * CGB * (DA // L), zb2, 0)
    nbase = sid * PPT
    for k in range(PPT // CGB):
        pltpu.sync_copy(msg2.at[0], acc_s.at[pl.ds(nbase + k * CGB, CGB)])
    rem = PPT - (PPT // CGB) * CGB
    pltpu.sync_copy(msg2.at[0, pl.ds(0, rem)],
                    acc_s.at[pl.ds(nbase + PPT - rem, rem)])
    plsc.subcore_barrier()

    def unpack_chunk(slot):
        for r in range(CGB // L):
            si, di = _unpack(pk2[slot, pl.ds(r * L, L)])
            if (r + 1) * L > CG:
                si = jnp.where(m8, si, 0)
                di = jnp.where(m8, di, 0)
            si2[slot, pl.ds(r * L, L)] = si
            di2[slot, pl.ds(r * L, L)] = di

    def idx_start(c, slot):
        pltpu.async_copy(pk_hbm.at[pl.ds(ebase + c * CG, CG)],
                         pk2.at[slot, pl.ds(0, CG)], s_i.at[slot])

    def idx_wait(slot):
        pltpu.make_async_copy(pk_hbm.at[pl.ds(ebase, CG)],
                              pk2.at[slot, pl.ds(0, CG)],
                              s_i.at[slot]).wait()

    def gather_start(slot):
        pltpu.async_copy(ff_hbm.at[si2.at[slot]], fb2.at[slot],
                         s_g.at[slot])

    def gather_wait(slot):
        pltpu.make_async_copy(ff_hbm.at[si2.at[slot]], fb2.at[slot],
                              s_g.at[slot]).wait()

    def scat_start(slot):
        pltpu.async_copy(msg2.at[slot], acc_s.at[di2.at[slot]],
                         s_s.at[slot], add=True)

    def scat_wait(slot):
        pltpu.make_async_copy(msg2.at[slot], acc_s.at[di2.at[slot]],
                              s_s.at[slot]).wait()

    idx_start(0, 0)
    idx_wait(0)
    unpack_chunk(0)
    gather_start(0)
    idx_start(1, 1)

    def chunk(c, _):
        b = jnp.bitwise_and(c, 1)
        nb = 1 - b

        @pl.when(c + 1 < NCHG)
        def _():
            idx_wait(nb)

            @pl.when(c >= 1)
            def _():
                scat_wait(nb)

            unpack_chunk(nb)
            gather_start(nb)

            @pl.when(c + 2 < NCHG)
            def _():
                idx_start(c + 2, b)

        gather_wait(b)

        def edge(i, _):
            lrow = jnp.zeros((L,), _f32)
            fsvs = []
            for h in range(H):
                fsv = fb2[b, i, pl.ds(h * L, L)]
                fsvs.append(fsv)
                v = fsv + fb2[b, i, pl.ds(D + h * L, L)]
                e = jnp.where(v >= 0.0, v, 0.2 * v)
                lg = jnp.sum(e * attn_v[h, :])
                lrow = jnp.where(lane == h,
                                 jnp.broadcast_to(lg, (L,)), lrow)
            exrow = jnp.exp(lrow)
            for h in range(H):
                exv = jnp.broadcast_to(exrow[h], (L,))
                msg2[b, i, pl.ds(h * L, L)] = fsvs[h] * exv
            msg2[b, i, pl.ds(D, L)] = exrow
            return 0

        lax.fori_loop(0, CG, edge, 0)
        scat_start(b)
        return 0

    lax.fori_loop(0, NCHG, chunk, 0)
    scat_wait((NCHG - 1) & 1)
    scat_wait((NCHG - 2) & 1)
    plsc.subcore_barrier()
    pltpu.sync_copy(acc_s.at[pl.ds(nbase, PPT)], out_hbm.at[cid, sid])


def _tc_xw_body(x_ref, w_ref, o_ref):
    o_ref[...] = jnp.dot(x_ref[...], w_ref[...],
                         preferred_element_type=_f32)


def _tc_scale_body(xw_ref, degs_ref, o_ref):
    deg = jnp.sum(degs_ref[...], axis=0, keepdims=True)
    no = lax.rsqrt(jnp.maximum(deg, 1.0))
    o_ref[...] = xw_ref[...] * jnp.transpose(no)


def _tc_post_gc_body(aggp_ref, degd_ref, bgc_ref, ws_ref, bs_ref, wd_ref,
                     bd_ref, h_ref, ff_ref):
    deg = jnp.sum(degd_ref[...], axis=0, keepdims=True)
    ni = jnp.transpose(lax.rsqrt(jnp.maximum(deg, 1.0)))
    agg = aggp_ref[0, :N] + aggp_ref[1, :N]
    h = jnp.maximum(agg * ni + bgc_ref[...][None, :], 0.0)
    h_ref[...] = h
    ff_ref[:, :D] = jnp.dot(h, ws_ref[...], preferred_element_type=_f32) \
        + bs_ref[...][None, :]
    ff_ref[:, D:] = jnp.dot(h, wd_ref[...], preferred_element_type=_f32) \
        + bd_ref[...][None, :]


def _gat_combine(gatp_ref, hprev_ref):
    U = gatp_ref[0, :N, :D] + gatp_ref[1, :N, :D]
    s8 = gatp_ref[0, :N, D:D + H] + gatp_ref[1, :N, D:D + H]
    row8 = lax.broadcasted_iota(jnp.int32, (H, D), 0)
    col = lax.broadcasted_iota(jnp.int32, (H, D), 1)
    k8 = (col // DH == row8).astype(_f32)
    den = jnp.dot(s8 + 1e-9, k8, preferred_element_type=_f32)
    return jnp.maximum(U / den + hprev_ref[...], 0.0)


def _tc_post_gat_body(gatp_ref, hprev_ref, ws_ref, bs_ref, wd_ref, bd_ref,
                      h_ref, ff_ref):
    hn = _gat_combine(gatp_ref, hprev_ref)
    h_ref[...] = hn
    ff_ref[:, :D] = jnp.dot(hn, ws_ref[...], preferred_element_type=_f32) \
        + bs_ref[...][None, :]
    ff_ref[:, D:] = jnp.dot(hn, wd_ref[...], preferred_element_type=_f32) \
        + bd_ref[...][None, :]


def _tc_final_body(gatp_ref, hprev_ref, wc1_ref, bc1_ref, wc2_ref, bc2_ref,
                   wc3_ref, bc3_ref, o_ref):
    hn = _gat_combine(gatp_ref, hprev_ref)
    hg = jnp.sum(hn, axis=0, keepdims=True)
    z = jnp.maximum(jnp.dot(hg, wc1_ref[...], preferred_element_type=_f32)
                    + bc1_ref[...][None, :], 0.0)
    z = jnp.maximum(jnp.dot(z, wc2_ref[...], preferred_element_type=_f32)
                    + bc2_ref[...][None, :], 0.0)
    o_ref[...] = jnp.dot(z, wc3_ref[...], preferred_element_type=_f32) \
        + bc3_ref[...][None, :]


def _sds(shape):
    return jax.ShapeDtypeStruct(shape, _f32)


def kernel(x, edge_index, W_gc, b_gc, Wsrc0, bsrc0, Wdst0, bdst0, attn0,
           Wsrc1, bsrc1, Wdst1, bdst1, attn1, Wsrc2, bsrc2, Wdst2, bdst2,
           attn2, Wc1, bc1, Wc2, bc2, Wc3, bc3):
    src = edge_index[0].astype(jnp.int32)
    dst = edge_index[1].astype(jnp.int32)
    pk = jnp.bitwise_or(src, lax.shift_left(dst, 16))

    degs, degd = _sc_degrees(pk)
    degs = degs.reshape(NW, N)
    degd = degd.reshape(NW, N)
    xw = pl.pallas_call(_tc_xw_body, out_shape=_sds((N, D)))(x, W_gc)
    xwn = pl.pallas_call(_tc_scale_body, out_shape=_sds((N, D)))(xw, degs)
    aggp = _sc_gc_agg(xwn, pk).reshape(NC, NP, D)
    h, ff = pl.pallas_call(
        _tc_post_gc_body,
        out_shape=[_sds((N, D)), _sds((N, 2 * D))],
    )(aggp, degd, b_gc, Wsrc0, bsrc0, Wdst0, bdst0)

    for (ws, bs, wd, bd, attn) in ((Wsrc1, bsrc1, Wdst1, bdst1, attn0),
                                   (Wsrc2, bsrc2, Wdst2, bdst2, attn1)):
        gp = _sc_gat(ff, attn, pk).reshape(NC, NP, DA)
        h, ff = pl.pallas_call(
            _tc_post_gat_body,
            out_shape=[_sds((N, D)), _sds((N, 2 * D))],
        )(gp, h, ws, bs, wd, bd)

    gp = _sc_gat(ff, attn2, pk).reshape(NC, NP, DA)
    return pl.pallas_call(
        _tc_final_body, out_shape=_sds((1, OUT)),
    )(gp, h, Wc1, bc1, Wc2, bc2, Wc3, bc3)
